# Initial kernel scaffold; baseline (speedup 1.0000x reference)
import jax
import jax.numpy as jnp
from jax.experimental import pallas as pl


def kernel(x, edge_index, edge_attr, edge_weights, W1, b1, W2, b2, W3, b3):
    raise NotImplementedError("write your pallas kernel here")


# speedup vs baseline: 8.7624x; 8.7624x over previous
import functools

import jax
import jax.numpy as jnp
from jax import lax
from jax.experimental import pallas as pl
from jax.experimental.pallas import tpu as pltpu
from jax.experimental.pallas import tpu_sc as plsc

N = 10000
E = 320000
F_IN = 128
H1 = 2 * F_IN
H2 = 3 * F_IN

NCORE = 2
NSUB = 16
NW = NCORE * NSUB
LANES = 16

B = 80
B1 = 2000
NROWCHUNK = N // B

_MESH = plsc.VectorSubcoreMesh(core_axis_name="c", subcore_axis_name="s")


def _zv():
    return jnp.zeros((LANES,), jnp.float32)


def _al8(v):
    return pl.multiple_of(v, 8)


def _n_row_chunks(s):
    return jnp.where(s < NROWCHUNK % NSUB,
                     NROWCHUNK // NSUB + 1, NROWCHUNK // NSUB)


def _zero_slab_and_barrier(slab, rows_v, s, fh):
    def zrow(i, _):
        for j in range(fh // LANES):
            rows_v[i, pl.ds(j * LANES, LANES)] = _zv()
        return 0

    lax.fori_loop(0, B, zrow, 0)

    def body(t, _):
        r0 = _al8((s + NSUB * t) * B)
        pltpu.sync_copy(rows_v, slab.at[pl.ds(r0, B)])
        return 0

    lax.fori_loop(0, _n_row_chunks(s), body, 0)
    plsc.subcore_barrier()


def _writeout(slab, out_hbm, row0, s):
    def body(t, _):
        r0 = _al8((s + NSUB * t) * B)
        pltpu.sync_copy(slab.at[pl.ds(r0, B)],
                        out_hbm.at[pl.ds(_al8(row0 + r0), B)])
        return 0

    lax.fori_loop(0, _n_row_chunks(s), body, 0)


def _make_agg_kernel(table_rows, split_edges_by_core):
    fh = 128
    per_tile = E // (NW if split_edges_by_core else NSUB)
    nchunk = per_tile // B

    @functools.partial(
        pl.kernel,
        out_type=jax.ShapeDtypeStruct((NCORE * N, fh), jnp.float32),
        mesh=_MESH,
        scratch_types=[
            pltpu.VMEM_SHARED((N, fh), jnp.float32),
            pltpu.VMEM((B,), jnp.int32),
            pltpu.VMEM((B,), jnp.int32),
            pltpu.VMEM((B, LANES), jnp.float32),
            pltpu.VMEM((B, fh), jnp.float32),
            pltpu.SemaphoreType.DMA,
        ],
    )
    def agg_kernel(tab_hbm, src_hbm, dst_hbm, ew_hbm, out_hbm,
                   slab, src_v, dst_v, ew_v, rows_v, sem):
        c = lax.axis_index("c")
        s = lax.axis_index("s")
        _zero_slab_and_barrier(slab, rows_v, s, fh)
        row0 = c * N
        if split_edges_by_core:
            base0 = (c * NSUB + s) * per_tile
        else:
            base0 = s * per_tile

        def chunk(k, _):
            base = _al8(base0 + k * B)
            pltpu.sync_copy(src_hbm.at[pl.ds(base, B)], src_v)
            pltpu.sync_copy(dst_hbm.at[pl.ds(base, B)], dst_v)
            pltpu.sync_copy(ew_hbm.at[pl.ds(base, B)], ew_v)
            if not split_edges_by_core:
                for t in range(B // LANES):
                    sl = pl.ds(t * LANES, LANES)
                    src_v[sl] = src_v[sl] + row0
            pltpu.async_copy(tab_hbm.at[src_v], rows_v, sem).wait()

            def scale(i, _):
                w = ew_v[i]
                for j in range(fh // LANES):
                    sl = pl.ds(j * LANES, LANES)
                    rows_v[i, sl] = rows_v[i, sl] * w
                return 0

            lax.fori_loop(0, B, scale, 0)
            pltpu.sync_copy(rows_v, slab.at[dst_v], add=True)
            return 0

        lax.fori_loop(0, nchunk, chunk, 0)
        plsc.subcore_barrier()
        _writeout(slab, out_hbm, row0, s)

    return agg_kernel


def _make_scalar_agg_kernel(gather_table):
    per_tile = E // NW
    nchunk = per_tile // B1

    scratch = [pltpu.VMEM((N,), jnp.float32),
               pltpu.VMEM((B1,), jnp.int32),
               pltpu.VMEM((B1,), jnp.float32)]
    if gather_table:
        scratch = ([pltpu.VMEM((N,), jnp.float32),
                    pltpu.VMEM((B1,), jnp.int32)] + scratch)

    @functools.partial(
        pl.kernel,
        out_type=jax.ShapeDtypeStruct((NW * N,), jnp.float32),
        mesh=_MESH,
        scratch_types=scratch,
        compiler_params=pltpu.CompilerParams(needs_layout_passes=False),
    )
    def scalar_kernel(*args):
        if gather_table:
            (tab_hbm, src_hbm, dst_hbm, ew_hbm, out_hbm,
             tab_v, src_v, part_v, dst_v, ew_v) = args
        else:
            dst_hbm, ew_hbm, out_hbm, part_v, dst_v, ew_v = args
        c = lax.axis_index("c")
        s = lax.axis_index("s")
        wid = c * NSUB + s

        def zp(i, _):
            part_v[pl.ds(i * LANES, LANES)] = _zv()
            return 0

        lax.fori_loop(0, N // LANES, zp, 0)
        if gather_table:
            pltpu.sync_copy(tab_hbm, tab_v)
        base0 = wid * per_tile

        def chunk(k, _):
            base = _al8(base0 + k * B1)
            if gather_table:
                pltpu.sync_copy(src_hbm.at[pl.ds(base, B1)], src_v)
            pltpu.sync_copy(dst_hbm.at[pl.ds(base, B1)], dst_v)
            pltpu.sync_copy(ew_hbm.at[pl.ds(base, B1)], ew_v)

            def step(t, _):
                sl = pl.ds(t * LANES, LANES)
                dv = dst_v[sl]
                wv = ew_v[sl]
                if gather_table:
                    wv = wv * plsc.load_gather(tab_v, [src_v[sl]])
                plsc.addupdate_scatter(part_v, [dv], wv)
                return 0

            lax.fori_loop(0, B1 // LANES, step, 0)
            return 0

        lax.fori_loop(0, nchunk, chunk, 0)
        pltpu.sync_copy(part_v, out_hbm.at[pl.ds(_al8(wid * N), N)])

    return scalar_kernel


_deg_call = _make_scalar_agg_kernel(False)
_zs_call = _make_scalar_agg_kernel(True)
_agg1_call = _make_agg_kernel(N, True)
_agg2_call = _make_agg_kernel(2 * N, False)


_R = 1000


def _prep_body(deg_ref, x_ref, dis_ref, hs1_ref):
    degsum = jnp.sum(deg_ref[...], axis=1)
    dis = lax.rsqrt(degsum + 1.0)
    dis_ref[...] = dis[:, None]
    hs1_ref[...] = dis[:, None] * x_ref[...]


def _l1_body(p_ref, hs_ref, dis_ref, w_ref, b_ref, out_ref):
    dis = dis_ref[...]
    t2 = dis * (p_ref[0] + p_ref[1] + hs_ref[...])
    h = jnp.maximum(jnp.dot(t2, w_ref[...],
                            preferred_element_type=jnp.float32) + b_ref[...],
                    0.0)
    dh = dis * h
    out_ref[...] = jnp.stack([dh[:, :H1 // 2], dh[:, H1 // 2:]], axis=0)


def _l2_body(aggs_ref, hs_ref, dis_ref, w2_ref, b2_ref, w3_ref, zs_ref):
    dis = dis_ref[...]
    t = dis[None] * (aggs_ref[...] + hs_ref[...])
    t2 = jnp.concatenate([t[0], t[1]], axis=1)
    h = jnp.maximum(jnp.dot(t2, w2_ref[...],
                            preferred_element_type=jnp.float32) + b2_ref[...],
                    0.0)
    zc = jnp.dot(h, w3_ref[...], preferred_element_type=jnp.float32)
    zs_ref[...] = (dis * zc)[:, 0:1]


def _fin_body(a3_ref, zs_ref, dis_ref, b3_ref, out_ref):
    a3 = jnp.sum(a3_ref[...], axis=1)
    out_ref[...] = (dis_ref[:, 0] * (a3 + zs_ref[:, 0]) + b3_ref[0, 0])[:, None]


def _rows_spec(shape):
    if len(shape) == 3:
        return pl.BlockSpec((shape[0], _R, shape[2]), lambda i: (0, i, 0))
    if shape[0] == N:
        return pl.BlockSpec((_R, shape[1]), lambda i: (i, 0))
    return pl.BlockSpec((shape[0], _R), lambda i: (0, i))


def _full_spec(shape):
    return pl.BlockSpec(shape, lambda i: tuple(0 for _ in shape))


def _prep_call(deg, x):
    return pl.pallas_call(
        _prep_body,
        grid=(N // _R,),
        in_specs=[_rows_spec((N, NW)), _rows_spec((N, F_IN))],
        out_specs=[_rows_spec((N, 1)), _rows_spec((N, F_IN))],
        out_shape=[jax.ShapeDtypeStruct((N, 1), jnp.float32),
                   jax.ShapeDtypeStruct((N, F_IN), jnp.float32)],
    )(deg, x)


def _l1_call(p, hs, dis, w, b):
    return pl.pallas_call(
        _l1_body,
        grid=(N // _R,),
        in_specs=[_rows_spec((2, N, F_IN)), _rows_spec((N, F_IN)),
                  _rows_spec((N, 1)), _full_spec((F_IN, H1)),
                  _full_spec((1, H1))],
        out_specs=_rows_spec((2, N, H1 // 2)),
        out_shape=jax.ShapeDtypeStruct((2, N, H1 // 2), jnp.float32),
    )(p, hs, dis, w, b)


def _l2_call(aggs, hs, dis, w2, b2, w3t):
    return pl.pallas_call(
        _l2_body,
        grid=(N // _R,),
        in_specs=[_rows_spec((2, N, H1 // 2)), _rows_spec((2, N, H1 // 2)),
                  _rows_spec((N, 1)), _full_spec((H1, H2)),
                  _full_spec((1, H2)), _full_spec((H2, LANES))],
        out_specs=_rows_spec((N, 1)),
        out_shape=jax.ShapeDtypeStruct((N, 1), jnp.float32),
    )(aggs, hs, dis, w2, b2, w3t)


def _fin_call(a3, zs, dis, b3):
    return pl.pallas_call(
        _fin_body,
        grid=(N // _R,),
        in_specs=[_rows_spec((N, NW)), _rows_spec((N, 1)),
                  _rows_spec((N, 1)), _full_spec((1, 1))],
        out_specs=_rows_spec((N, 1)),
        out_shape=jax.ShapeDtypeStruct((N, 1), jnp.float32),
    )(a3, zs, dis, b3)


def kernel(x, edge_index, edge_attr, edge_weights, W1, b1, W2, b2, W3, b3):
    del edge_attr
    src = edge_index[0]
    dst = edge_index[1]
    ew = edge_weights

    ew16 = jnp.broadcast_to(ew[:, None], (E, LANES))

    deg = _deg_call(dst, ew).reshape(NW, N).T
    dis, hs1 = _prep_call(deg, x)
    p1 = _agg1_call(hs1, src, dst, ew16)
    hs2 = _l1_call(p1.reshape(2, N, F_IN), hs1, dis,
                   W1, b1.reshape(1, H1))
    aggs2 = _agg2_call(hs2.reshape(2 * N, H1 // 2), src, dst, ew16)
    zs = _l2_call(aggs2.reshape(2, N, H1 // 2), hs2, dis,
                  W2, b2.reshape(1, H2), jnp.tile(W3, (1, LANES)))
    a3 = _zs_call(zs.reshape(N), src, dst, ew).reshape(NW, N).T
    out = _fin_call(a3, zs, dis, b3.reshape(1, 1))
    return out
